# baseline (speedup 1.0000x reference)
import functools

import jax
import jax.numpy as jnp
import numpy as np
from jax.experimental import pallas as pl
from jax.experimental.pallas import tpu as pltpu


def _log_softmax(logits):
    m = jnp.max(logits, axis=1, keepdims=True)
    z = logits - m
    lse = jnp.log(jnp.sum(jnp.exp(z), axis=1, keepdims=True))
    return z - lse


def _phase_a_kernel(x_ref, w1_ref, b1_ref, w2_ref, b2_ref, adj_ref,
                    s2_out_ref, part_ref, sup_s, s2_s, pend_s,
                    *, bm, nb, n, gb, pmax):
    i = pl.program_id(0)

    @pl.when(i == 0)
    def _():
        sup_s[...] = jnp.dot(x_ref[...], w1_ref[...],
                             preferred_element_type=jnp.float32)
        s2_s[...] = jnp.zeros_like(s2_s)

    h = jnp.dot(adj_ref[...], sup_s[...],
                preferred_element_type=jnp.float32)
    h = jnp.maximum(h + b1_ref[...], 0.0)
    s2 = jnp.dot(h, w2_ref[...], preferred_element_type=jnp.float32)
    row = jax.lax.broadcasted_iota(jnp.int32, s2.shape, 0)
    s2 = jnp.where(row < n - i * bm, s2, 0.0)

    if pmax > 0:
        @pl.when(i >= gb)
        def _():
            part_ref[...] = jnp.dot(
                adj_ref[:, pl.ds(0, pmax * gb * bm)],
                s2_s[pl.ds(0, pmax * gb * bm), :],
                preferred_element_type=jnp.float32) + b2_ref[...]

    pend_s[pl.ds(jax.lax.rem(i, gb) * bm, bm), :] = s2

    @pl.when((jax.lax.rem(i, gb) == gb - 1) & (i < pmax * gb))
    def _():
        s2_s[pl.ds((i - (gb - 1)) * bm, gb * bm), :] = pend_s[...]

    s2_out_ref[...] = s2


def _chunk_kernel(ri_ref, ci_ref, fi_ref, fe_ref, adj_ref, s2_ref,
                  part_ref, b2_ref, o_ref):
    t = pl.program_id(0)
    contrib = jnp.dot(adj_ref[...], s2_ref[...],
                      preferred_element_type=jnp.float32)

    @pl.when(fi_ref[t] == 1)
    def _():
        o_ref[...] = b2_ref[...] + contrib

    @pl.when(fi_ref[t] == 2)
    def _():
        o_ref[...] = part_ref[...] + contrib

    @pl.when(fi_ref[t] == 0)
    def _():
        o_ref[...] += contrib

    @pl.when(fe_ref[t] == 1)
    def _():
        o_ref[...] = _log_softmax(o_ref[...])


def kernel(x, adj, W1, b1, W2, b2):
    n, f_in = x.shape
    h_dim = W1.shape[1]
    c_dim = W2.shape[1]
    bm = 512
    gb = 10
    nb = -(-n // bm)
    wc = gb * bm
    nc = -(-n // wc)
    pmax = nc - 1

    b1_2d = b1.reshape(1, h_dim)
    b2_2d = b2.reshape(1, c_dim)

    s2_hbm, partial = pl.pallas_call(
        functools.partial(_phase_a_kernel, bm=bm, nb=nb, n=n, gb=gb,
                          pmax=pmax),
        grid=(nb,),
        in_specs=[
            pl.BlockSpec((n, f_in), lambda i: (0, 0)),
            pl.BlockSpec((f_in, h_dim), lambda i: (0, 0)),
            pl.BlockSpec((1, h_dim), lambda i: (0, 0)),
            pl.BlockSpec((h_dim, c_dim), lambda i: (0, 0)),
            pl.BlockSpec((1, c_dim), lambda i: (0, 0)),
            pl.BlockSpec((bm, n), lambda i: (i, 0)),
        ],
        out_specs=[
            pl.BlockSpec((bm, c_dim), lambda i: (i, 0)),
            pl.BlockSpec((bm, c_dim), lambda i: (i, 0)),
        ],
        scratch_shapes=[
            pltpu.VMEM((n, h_dim), jnp.float32),
            pltpu.VMEM((nb * bm, c_dim), jnp.float32),
            pltpu.VMEM((gb * bm, c_dim), jnp.float32),
        ],
        out_shape=[
            jax.ShapeDtypeStruct((nb * bm, c_dim), jnp.float32),
            jax.ShapeDtypeStruct((n, c_dim), jnp.float32),
        ],
        compiler_params=pltpu.CompilerParams(
            dimension_semantics=("arbitrary",)),
    )(x, W1, b1_2d, W2, b2_2d, adj)

    ri, ci, fi, fe = [], [], [], []
    for i in range(nb):
        q0 = min(i // gb, pmax)
        for j, c in enumerate(range(q0, nc)):
            ri.append(i)
            ci.append(c)
            fi.append((1 if q0 == 0 else 2) if j == 0 else 0)
            fe.append(1 if c == nc - 1 else 0)
    ri_t = jnp.asarray(np.array(ri, dtype=np.int32))
    ci_t = jnp.asarray(np.array(ci, dtype=np.int32))
    fi_t = jnp.asarray(np.array(fi, dtype=np.int32))
    fe_t = jnp.asarray(np.array(fe, dtype=np.int32))
    n_steps = ri_t.shape[0]

    out = pl.pallas_call(
        _chunk_kernel,
        grid_spec=pltpu.PrefetchScalarGridSpec(
            num_scalar_prefetch=4,
            grid=(n_steps,),
            in_specs=[
                pl.BlockSpec((bm, wc),
                             lambda t, ri, ci, fi, fe: (ri[t], ci[t])),
                pl.BlockSpec((wc, c_dim),
                             lambda t, ri, ci, fi, fe: (ci[t], 0)),
                pl.BlockSpec((bm, c_dim),
                             lambda t, ri, ci, fi, fe: (ri[t], 0)),
                pl.BlockSpec((1, c_dim), lambda t, ri, ci, fi, fe: (0, 0)),
            ],
            out_specs=pl.BlockSpec((bm, c_dim),
                                   lambda t, ri, ci, fi, fe: (ri[t], 0)),
        ),
        out_shape=jax.ShapeDtypeStruct((n, c_dim), jnp.float32),
        compiler_params=pltpu.CompilerParams(
            dimension_semantics=("arbitrary",)),
    )(ri_t, ci_t, fi_t, fe_t, adj, s2_hbm, partial, b2_2d)

    return out


# speedup vs baseline: 1.0108x; 1.0108x over previous
import functools

import jax
import jax.numpy as jnp
import numpy as np
from jax.experimental import pallas as pl
from jax.experimental.pallas import tpu as pltpu


def _log_softmax(logits):
    m = jnp.max(logits, axis=1, keepdims=True)
    z = logits - m
    lse = jnp.log(jnp.sum(jnp.exp(z), axis=1, keepdims=True))
    return z - lse


def _phase_a_kernel(x_ref, w1_ref, b1_ref, w2_ref, b2_ref, adj_ref,
                    s2_out_ref, part_ref, sup_s, s2_s, pend_s,
                    *, bm, nb, n, gb, pmax):
    i = pl.program_id(0)

    @pl.when(i == 0)
    def _():
        sup_s[...] = jnp.dot(x_ref[...], w1_ref[...],
                             preferred_element_type=jnp.float32)
        s2_s[...] = jnp.zeros_like(s2_s)

    h = jnp.dot(adj_ref[...], sup_s[...],
                preferred_element_type=jnp.float32)
    h = jnp.maximum(h + b1_ref[...], 0.0)
    s2 = jnp.dot(h, w2_ref[...], preferred_element_type=jnp.float32)
    row = jax.lax.broadcasted_iota(jnp.int32, s2.shape, 0)
    s2 = jnp.where(row < n - i * bm, s2, 0.0)

    if pmax > 0:
        @pl.when(i >= gb)
        def _():
            part_ref[...] = jnp.dot(
                adj_ref[:, pl.ds(0, pmax * gb * bm)],
                s2_s[pl.ds(0, pmax * gb * bm), :],
                preferred_element_type=jnp.float32) + b2_ref[...]

    pend_s[pl.ds(jax.lax.rem(i, gb) * bm, bm), :] = s2

    @pl.when((jax.lax.rem(i, gb) == gb - 1) & (i < pmax * gb))
    def _():
        s2_s[pl.ds((i - (gb - 1)) * bm, gb * bm), :] = pend_s[...]

    s2_out_ref[...] = s2


def _chunk_kernel(ri_ref, ci_ref, fi_ref, fe_ref, adj_ref, s2_ref,
                  part_ref, b2_ref, o_ref):
    t = pl.program_id(0)
    contrib = jnp.dot(adj_ref[...], s2_ref[...],
                      preferred_element_type=jnp.float32)

    @pl.when(fi_ref[t] == 1)
    def _():
        o_ref[...] = b2_ref[...] + contrib

    @pl.when(fi_ref[t] == 2)
    def _():
        o_ref[...] = part_ref[...] + contrib

    @pl.when(fi_ref[t] == 0)
    def _():
        o_ref[...] += contrib

    @pl.when(fe_ref[t] == 1)
    def _():
        o_ref[...] = _log_softmax(o_ref[...])


def kernel(x, adj, W1, b1, W2, b2):
    n, f_in = x.shape
    h_dim = W1.shape[1]
    c_dim = W2.shape[1]
    bm = 512
    gb = 5
    nb = -(-n // bm)
    wc = gb * bm
    nc = -(-n // wc)
    pmax = nc - 1

    b1_2d = b1.reshape(1, h_dim)
    b2_2d = b2.reshape(1, c_dim)

    s2_hbm, partial = pl.pallas_call(
        functools.partial(_phase_a_kernel, bm=bm, nb=nb, n=n, gb=gb,
                          pmax=pmax),
        grid=(nb,),
        in_specs=[
            pl.BlockSpec((n, f_in), lambda i: (0, 0)),
            pl.BlockSpec((f_in, h_dim), lambda i: (0, 0)),
            pl.BlockSpec((1, h_dim), lambda i: (0, 0)),
            pl.BlockSpec((h_dim, c_dim), lambda i: (0, 0)),
            pl.BlockSpec((1, c_dim), lambda i: (0, 0)),
            pl.BlockSpec((bm, n), lambda i: (i, 0)),
        ],
        out_specs=[
            pl.BlockSpec((bm, c_dim), lambda i: (i, 0)),
            pl.BlockSpec((bm, c_dim), lambda i: (i, 0)),
        ],
        scratch_shapes=[
            pltpu.VMEM((n, h_dim), jnp.float32),
            pltpu.VMEM((nb * bm, c_dim), jnp.float32),
            pltpu.VMEM((gb * bm, c_dim), jnp.float32),
        ],
        out_shape=[
            jax.ShapeDtypeStruct((nb * bm, c_dim), jnp.float32),
            jax.ShapeDtypeStruct((n, c_dim), jnp.float32),
        ],
        compiler_params=pltpu.CompilerParams(
            dimension_semantics=("arbitrary",)),
    )(x, W1, b1_2d, W2, b2_2d, adj)

    ri, ci, fi, fe = [], [], [], []
    for i in range(nb):
        q0 = min(i // gb, pmax)
        for j, c in enumerate(range(q0, nc)):
            ri.append(i)
            ci.append(c)
            fi.append((1 if q0 == 0 else 2) if j == 0 else 0)
            fe.append(1 if c == nc - 1 else 0)
    ri_t = jnp.asarray(np.array(ri, dtype=np.int32))
    ci_t = jnp.asarray(np.array(ci, dtype=np.int32))
    fi_t = jnp.asarray(np.array(fi, dtype=np.int32))
    fe_t = jnp.asarray(np.array(fe, dtype=np.int32))
    n_steps = ri_t.shape[0]

    out = pl.pallas_call(
        _chunk_kernel,
        grid_spec=pltpu.PrefetchScalarGridSpec(
            num_scalar_prefetch=4,
            grid=(n_steps,),
            in_specs=[
                pl.BlockSpec((bm, wc),
                             lambda t, ri, ci, fi, fe: (ri[t], ci[t])),
                pl.BlockSpec((wc, c_dim),
                             lambda t, ri, ci, fi, fe: (ci[t], 0)),
                pl.BlockSpec((bm, c_dim),
                             lambda t, ri, ci, fi, fe: (ri[t], 0)),
                pl.BlockSpec((1, c_dim), lambda t, ri, ci, fi, fe: (0, 0)),
            ],
            out_specs=pl.BlockSpec((bm, c_dim),
                                   lambda t, ri, ci, fi, fe: (ri[t], 0)),
        ),
        out_shape=jax.ShapeDtypeStruct((n, c_dim), jnp.float32),
        compiler_params=pltpu.CompilerParams(
            dimension_semantics=("arbitrary",)),
    )(ri_t, ci_t, fi_t, fe_t, adj, s2_hbm, partial, b2_2d)

    return out
